# baseline (speedup 1.0000x reference)
import functools

import jax
import jax.numpy as jnp
from jax import lax
from jax.experimental import pallas as pl
from jax.experimental.pallas import tpu as pltpu
from jax.experimental.pallas import tpu_sc as plsc

TAU = 0.13
ALPHA = 0.7

NC = 2
NS = 16
NW = NC * NS
L = 16

S_SC = 1536
LOG2E = 1.4426950408889634


def _take16(v, idx):
    return lax.gather(
        v, idx[:, None],
        dimension_numbers=lax.GatherDimensionNumbers(
            offset_dims=(), collapsed_slice_dims=(0,), start_index_map=(0,)),
        slice_sizes=(1,),
        mode=lax.GatherScatterMode.PROMISE_IN_BOUNDS)


def _lanesum(v):
    iot = lax.iota(jnp.int32, L)
    for s in (8, 4, 2, 1):
        v = v + _take16(v, jnp.bitwise_xor(iot, s))
    return v


def _make_sc_kernel(S, V):
    RPW = S // NW
    U = 5
    NU = V // (L * U)

    mesh = plsc.VectorSubcoreMesh(core_axis_name="c", subcore_axis_name="s")

    @functools.partial(
        pl.kernel,
        mesh=mesh,
        out_type=jax.ShapeDtypeStruct((NW, L), jnp.float32),
        scratch_types=[
            pltpu.VMEM((RPW + L,), jnp.int32),
            pltpu.VMEM((RPW + L,), jnp.float32),
            pltpu.VMEM((4, V), jnp.float32),
            pltpu.VMEM((2, 4, V), jnp.float32),
            pltpu.VMEM((L,), jnp.float32),
            pltpu.SemaphoreType.DMA,
            pltpu.SemaphoreType.DMA,
            pltpu.SemaphoreType.DMA,
            pltpu.SemaphoreType.DMA,
            pltpu.SemaphoreType.DMA,
            pltpu.SemaphoreType.DMA,
        ],
    )
    def sc_kernel(inp_hbm, tgt_hbm, mask_hbm, sim_hbm, out_hbm,
                  tgt_v, mask_v, sim_buf, inp_buf, res_v,
                  sem_s0, sem_s1, sem_s2, sem_s3, sem_i0, sem_i1):
        wid = lax.axis_index("s") * NC + lax.axis_index("c")
        base = wid * RPW
        iotav = lax.iota(jnp.int32, L)

        pltpu.sync_copy(tgt_hbm.at[pl.ds(base, RPW)], tgt_v.at[pl.ds(0, RPW)])
        pltpu.sync_copy(mask_hbm.at[pl.ds(base, RPW)],
                        mask_v.at[pl.ds(0, RPW)])

        def scalar_at(ref, i):
            return ref[pl.ds(i, L)][0]

        sem_s = (sem_s0, sem_s1, sem_s2, sem_s3)
        sem_i = (sem_i0, sem_i1)

        def sim_copy(r, b):
            return pltpu.make_async_copy(sim_hbm.at[scalar_at(tgt_v, r)],
                                         sim_buf.at[b], sem_s[b])

        def inpblk_copy(g, slot):
            return pltpu.make_async_copy(inp_hbm.at[pl.ds(base + g * 4, 4)],
                                         inp_buf.at[slot], sem_i[slot])

        for slot in range(2):
            inpblk_copy(slot, slot).start()
        for b in range(4):
            sim_copy(b, b).start()

        NBLK = RPW // 4

        def gg_body(gg, carry):
            for gslot in range(2):
                g = gg * 2 + gslot
                inpblk_copy(g, gslot).wait()
                acc_out, acc_ml, acc_m = carry
                for b in range(4):
                    r = g * 4 + b
                    sim_copy(r, b).wait()

                    def chunk_body(c, acc):
                        accs = list(acc)
                        off0 = c * (L * U)
                        for u in range(U):
                            off = off0 + u * L
                            s = jnp.exp(sim_buf[b, pl.ds(off, L)]
                                        * (1.0 / TAU))
                            accs[2 * u] = accs[2 * u] + s
                            accs[2 * u + 1] = (accs[2 * u + 1]
                                               + s * inp_buf[gslot, b,
                                                             pl.ds(off, L)])
                        return tuple(accs)

                    z = jnp.zeros((L,), jnp.float32)
                    accs = lax.fori_loop(0, NU, chunk_body, (z,) * (2 * U))
                    d_acc = accs[0]
                    x_acc = accs[1]
                    for u in range(1, U):
                        d_acc = d_acc + accs[2 * u]
                        x_acc = x_acc + accs[2 * u + 1]
                    c_vec = _lanesum(x_acc) / _lanesum(d_acc)

                    tgt_s = scalar_at(tgt_v, r)
                    toff = (tgt_s // L) * L
                    logit_vec = _lanesum(
                        jnp.where(iotav == tgt_s - toff,
                                  inp_buf[gslot, b, pl.ds(toff, L)], 0.0))

                    m_vec = jnp.full((L,), scalar_at(mask_v, r), jnp.float32)
                    acc_out = acc_out + m_vec * c_vec
                    acc_ml = acc_ml + m_vec * logit_vec
                    acc_m = acc_m + m_vec

                    @pl.when(r + 4 < RPW)
                    def _():
                        sim_copy(r + 4, b).start()
                carry = (acc_out, acc_ml, acc_m)

                @pl.when(g + 2 < NBLK)
                def _():
                    inpblk_copy(g + 2, gslot).start()
            return carry

        zero = jnp.zeros((L,), jnp.float32)
        acc_out, acc_ml, acc_m = lax.fori_loop(0, RPW // 8, gg_body,
                                               (zero, zero, zero))

        res = jnp.where(iotav == 0, -acc_ml,
                        jnp.where(iotav == 1, -acc_out,
                                  jnp.where(iotav == 2, acc_m, 0.0)))
        res_v[...] = res
        pltpu.sync_copy(res_v, out_hbm.at[wid])

    return sc_kernel


def _make_tc_kernel(BT, V, S):
    NT = BT - S
    RB = 16
    NG = NT // RB

    def tc_body(tgt_ref, mask_ref, sim_hbm, inp_ref, out_ref, simscr,
                sem0, sem1):
        g = pl.program_id(0)
        sems = (sem0, sem1)

        def start_rows(gg, slot):
            for j in range(RB):
                row = tgt_ref[S + gg * RB + j]
                pltpu.make_async_copy(sim_hbm.at[row],
                                      simscr.at[slot, j],
                                      sems[slot]).start()

        def wait_rows(slot):
            pltpu.make_async_copy(sim_hbm.at[pl.ds(0, RB)],
                                  simscr.at[slot],
                                  sems[slot]).wait()

        @pl.when(g == 0)
        def _():
            start_rows(0, 0)

        @pl.when((g + 1 < NG) & (g % 2 == 0))
        def _():
            start_rows(g + 1, 1)

        @pl.when((g + 1 < NG) & (g % 2 == 1))
        def _():
            start_rows(g + 1, 0)

        @pl.when(g == 0)
        def _():
            out_ref[...] = jnp.zeros_like(out_ref)

        @pl.when(g % 2 == 0)
        def _():
            wait_rows(0)

        @pl.when(g % 2 == 1)
        def _():
            wait_rows(1)

        par = g % 2
        sim_block = simscr[pl.ds(par, 1)][0]
        inp_block = inp_ref[...]
        s = jnp.exp2(sim_block * (LOG2E / TAU))
        denom = jnp.sum(s, axis=1)
        dot = jnp.sum(inp_block * s, axis=1)
        m = jnp.stack([mask_ref[S + g * RB + j] for j in range(RB)])
        tgt_col = jnp.stack([tgt_ref[S + g * RB + j]
                             for j in range(RB)])[:, None]
        lanes = lax.broadcasted_iota(jnp.int32, (RB, V), 1)
        logit = jnp.sum(jnp.where(lanes == tgt_col, inp_block, 0.0),
                        axis=1)

        ml_c = -jnp.sum(m * logit)
        out_c = -jnp.sum(m * dot / denom)
        m_c = jnp.sum(m)

        lane = lax.broadcasted_iota(jnp.int32, (8, 128), 1)
        sub = lax.broadcasted_iota(jnp.int32, (8, 128), 0)
        contrib = jnp.where(
            (sub == 0) & (lane == 0), ml_c,
            jnp.where((sub == 0) & (lane == 1), out_c,
                      jnp.where((sub == 0) & (lane == 2), m_c, 0.0)))
        out_ref[...] += contrib

    grid_spec = pltpu.PrefetchScalarGridSpec(
        num_scalar_prefetch=2,
        grid=(NG,),
        in_specs=[
            pl.BlockSpec(memory_space=pltpu.MemorySpace.HBM),
            pl.BlockSpec((RB, V), lambda g, tgt, msk: (S // RB + g, 0)),
        ],
        out_specs=pl.BlockSpec((8, 128), lambda g, tgt, msk: (0, 0)),
        scratch_shapes=[
            pltpu.VMEM((2, RB, V), jnp.float32),
            pltpu.SemaphoreType.DMA,
            pltpu.SemaphoreType.DMA,
        ],
    )
    return pl.pallas_call(
        tc_body,
        grid_spec=grid_spec,
        out_shape=jax.ShapeDtypeStruct((8, 128), jnp.float32),
        compiler_params=pltpu.CompilerParams(
            dimension_semantics=("arbitrary",)),
    )


@jax.jit
def kernel(input, target, mask, sim_matrix):
    B, T, V = input.shape
    BT = B * T
    inp2 = input.reshape(BT, V)
    tgt = target.reshape(BT).astype(jnp.int32)
    mask1 = mask.reshape(BT)

    sc_partials = _make_sc_kernel(S_SC, V)(inp2, tgt, mask1, sim_matrix)
    tc_partials = _make_tc_kernel(BT, V, S_SC)(tgt, mask1, sim_matrix, inp2)

    ml_sum = jnp.sum(sc_partials[:, 0]) + tc_partials[0, 0]
    out_sum = jnp.sum(sc_partials[:, 1]) + tc_partials[0, 1]
    msum = jnp.sum(sc_partials[:, 2]) + tc_partials[0, 2]
    ml_output = ml_sum / msum
    output = ALPHA * (out_sum / msum) + (1.0 - ALPHA) * ml_output
    return (ml_output, output)


# speedup vs baseline: 1.0153x; 1.0153x over previous
import functools

import jax
import jax.numpy as jnp
from jax import lax
from jax.experimental import pallas as pl
from jax.experimental.pallas import tpu as pltpu
from jax.experimental.pallas import tpu_sc as plsc

TAU = 0.13
ALPHA = 0.7

NC = 2
NS = 16
NW = NC * NS
L = 16

S_SC = 1536
LOG2E = 1.4426950408889634


def _take16(v, idx):
    return lax.gather(
        v, idx[:, None],
        dimension_numbers=lax.GatherDimensionNumbers(
            offset_dims=(), collapsed_slice_dims=(0,), start_index_map=(0,)),
        slice_sizes=(1,),
        mode=lax.GatherScatterMode.PROMISE_IN_BOUNDS)


def _lanesum(v):
    iot = lax.iota(jnp.int32, L)
    for s in (8, 4, 2, 1):
        v = v + _take16(v, jnp.bitwise_xor(iot, s))
    return v


def _make_sc_kernel(S, V):
    RPW = S // NW
    U = 5
    NU = V // (L * U)

    mesh = plsc.VectorSubcoreMesh(core_axis_name="c", subcore_axis_name="s")

    @functools.partial(
        pl.kernel,
        mesh=mesh,
        out_type=jax.ShapeDtypeStruct((NW, L), jnp.float32),
        scratch_types=[
            pltpu.VMEM((RPW + L,), jnp.int32),
            pltpu.VMEM((RPW + L,), jnp.float32),
            pltpu.VMEM((4, V), jnp.float32),
            pltpu.VMEM((4, V), jnp.float32),
            pltpu.VMEM((L,), jnp.float32),
            pltpu.SemaphoreType.DMA,
            pltpu.SemaphoreType.DMA,
            pltpu.SemaphoreType.DMA,
            pltpu.SemaphoreType.DMA,
            pltpu.SemaphoreType.DMA,
            pltpu.SemaphoreType.DMA,
            pltpu.SemaphoreType.DMA,
            pltpu.SemaphoreType.DMA,
        ],
    )
    def sc_kernel(inp_hbm, tgt_hbm, mask_hbm, sim_hbm, out_hbm,
                  tgt_v, mask_v, sim_buf, inp_buf, res_v,
                  sem_s0, sem_s1, sem_s2, sem_s3, sem_i0, sem_i1,
                  sem_i2, sem_i3):
        wid = lax.axis_index("s") * NC + lax.axis_index("c")
        base = wid * RPW
        iotav = lax.iota(jnp.int32, L)

        pltpu.sync_copy(tgt_hbm.at[pl.ds(base, RPW)], tgt_v.at[pl.ds(0, RPW)])
        pltpu.sync_copy(mask_hbm.at[pl.ds(base, RPW)],
                        mask_v.at[pl.ds(0, RPW)])

        def scalar_at(ref, i):
            return ref[pl.ds(i, L)][0]

        sem_s = (sem_s0, sem_s1, sem_s2, sem_s3)
        sem_i = (sem_i0, sem_i1, sem_i2, sem_i3)

        def sim_copy(r, b):
            return pltpu.make_async_copy(sim_hbm.at[scalar_at(tgt_v, r)],
                                         sim_buf.at[b], sem_s[b])

        def inp_copy(r, b):
            return pltpu.make_async_copy(inp_hbm.at[base + r],
                                         inp_buf.at[b], sem_i[b])

        for b in range(4):
            sim_copy(b, b).start()
            inp_copy(b, b).start()

        def g_body(g, carry):
            acc_out, acc_ml, acc_m = carry
            for b in range(4):
                r = g * 4 + b
                sim_copy(r, b).wait()
                inp_copy(r, b).wait()

                def chunk_body(c, acc):
                    accs = list(acc)
                    off0 = c * (L * U)
                    for u in range(U):
                        off = off0 + u * L
                        s = jnp.exp(sim_buf[b, pl.ds(off, L)] * (1.0 / TAU))
                        accs[2 * u] = accs[2 * u] + s
                        accs[2 * u + 1] = (accs[2 * u + 1]
                                           + s * inp_buf[b, pl.ds(off, L)])
                    return tuple(accs)

                z = jnp.zeros((L,), jnp.float32)
                accs = lax.fori_loop(0, NU, chunk_body, (z,) * (2 * U))
                d_acc = accs[0]
                x_acc = accs[1]
                for u in range(1, U):
                    d_acc = d_acc + accs[2 * u]
                    x_acc = x_acc + accs[2 * u + 1]
                c_vec = _lanesum(x_acc) / _lanesum(d_acc)

                tgt_s = scalar_at(tgt_v, r)
                toff = (tgt_s // L) * L
                logit_vec = _lanesum(
                    jnp.where(iotav == tgt_s - toff,
                              inp_buf[b, pl.ds(toff, L)], 0.0))

                m_vec = jnp.full((L,), scalar_at(mask_v, r), jnp.float32)
                acc_out = acc_out + m_vec * c_vec
                acc_ml = acc_ml + m_vec * logit_vec
                acc_m = acc_m + m_vec

                @pl.when(r + 4 < RPW)
                def _():
                    sim_copy(r + 4, b).start()
                    inp_copy(r + 4, b).start()
            return (acc_out, acc_ml, acc_m)

        zero = jnp.zeros((L,), jnp.float32)
        acc_out, acc_ml, acc_m = lax.fori_loop(0, RPW // 4, g_body,
                                               (zero, zero, zero))

        res = jnp.where(iotav == 0, -acc_ml,
                        jnp.where(iotav == 1, -acc_out,
                                  jnp.where(iotav == 2, acc_m, 0.0)))
        res_v[...] = res
        pltpu.sync_copy(res_v, out_hbm.at[wid])

    return sc_kernel


def _make_tc_kernel(BT, V, S):
    NT = BT - S
    RB = 16
    NG = NT // RB

    def tc_body(tgt_ref, mask_ref, sim_hbm, inp_ref, out_ref, simscr,
                sem0, sem1):
        g = pl.program_id(0)
        sems = (sem0, sem1)

        def start_rows(gg, slot):
            for j in range(RB):
                row = tgt_ref[S + gg * RB + j]
                pltpu.make_async_copy(sim_hbm.at[row],
                                      simscr.at[slot, j],
                                      sems[slot]).start()

        def wait_rows(slot):
            pltpu.make_async_copy(sim_hbm.at[pl.ds(0, RB)],
                                  simscr.at[slot],
                                  sems[slot]).wait()

        @pl.when(g == 0)
        def _():
            start_rows(0, 0)

        @pl.when((g + 1 < NG) & (g % 2 == 0))
        def _():
            start_rows(g + 1, 1)

        @pl.when((g + 1 < NG) & (g % 2 == 1))
        def _():
            start_rows(g + 1, 0)

        @pl.when(g == 0)
        def _():
            out_ref[...] = jnp.zeros_like(out_ref)

        @pl.when(g % 2 == 0)
        def _():
            wait_rows(0)

        @pl.when(g % 2 == 1)
        def _():
            wait_rows(1)

        par = g % 2
        sim_block = simscr[pl.ds(par, 1)][0]
        inp_block = inp_ref[...]
        s = jnp.exp2(sim_block * (LOG2E / TAU))
        denom = jnp.sum(s, axis=1)
        dot = jnp.sum(inp_block * s, axis=1)
        m = jnp.stack([mask_ref[S + g * RB + j] for j in range(RB)])
        tgt_col = jnp.stack([tgt_ref[S + g * RB + j]
                             for j in range(RB)])[:, None]
        lanes = lax.broadcasted_iota(jnp.int32, (RB, V), 1)
        logit = jnp.sum(jnp.where(lanes == tgt_col, inp_block, 0.0),
                        axis=1)

        ml_c = -jnp.sum(m * logit)
        out_c = -jnp.sum(m * dot / denom)
        m_c = jnp.sum(m)

        lane = lax.broadcasted_iota(jnp.int32, (8, 128), 1)
        sub = lax.broadcasted_iota(jnp.int32, (8, 128), 0)
        contrib = jnp.where(
            (sub == 0) & (lane == 0), ml_c,
            jnp.where((sub == 0) & (lane == 1), out_c,
                      jnp.where((sub == 0) & (lane == 2), m_c, 0.0)))
        out_ref[...] += contrib

    grid_spec = pltpu.PrefetchScalarGridSpec(
        num_scalar_prefetch=2,
        grid=(NG,),
        in_specs=[
            pl.BlockSpec(memory_space=pltpu.MemorySpace.HBM),
            pl.BlockSpec((RB, V), lambda g, tgt, msk: (S // RB + g, 0)),
        ],
        out_specs=pl.BlockSpec((8, 128), lambda g, tgt, msk: (0, 0)),
        scratch_shapes=[
            pltpu.VMEM((2, RB, V), jnp.float32),
            pltpu.SemaphoreType.DMA,
            pltpu.SemaphoreType.DMA,
        ],
    )
    return pl.pallas_call(
        tc_body,
        grid_spec=grid_spec,
        out_shape=jax.ShapeDtypeStruct((8, 128), jnp.float32),
        compiler_params=pltpu.CompilerParams(
            dimension_semantics=("arbitrary",)),
    )


@jax.jit
def kernel(input, target, mask, sim_matrix):
    B, T, V = input.shape
    BT = B * T
    inp2 = input.reshape(BT, V)
    tgt = target.reshape(BT).astype(jnp.int32)
    mask1 = mask.reshape(BT)

    sc_partials = _make_sc_kernel(S_SC, V)(inp2, tgt, mask1, sim_matrix)
    tc_partials = _make_tc_kernel(BT, V, S_SC)(tgt, mask1, sim_matrix, inp2)

    ml_sum = jnp.sum(sc_partials[:, 0]) + tc_partials[0, 0]
    out_sum = jnp.sum(sc_partials[:, 1]) + tc_partials[0, 1]
    msum = jnp.sum(sc_partials[:, 2]) + tc_partials[0, 2]
    ml_output = ml_sum / msum
    output = ALPHA * (out_sum / msum) + (1.0 - ALPHA) * ml_output
    return (ml_output, output)


# speedup vs baseline: 1.0245x; 1.0091x over previous
import functools

import jax
import jax.numpy as jnp
from jax import lax
from jax.experimental import pallas as pl
from jax.experimental.pallas import tpu as pltpu
from jax.experimental.pallas import tpu_sc as plsc

TAU = 0.13
ALPHA = 0.7

NC = 2
NS = 16
NW = NC * NS
L = 16

S_SC = 1536
LOG2E = 1.4426950408889634


def _take16(v, idx):
    return lax.gather(
        v, idx[:, None],
        dimension_numbers=lax.GatherDimensionNumbers(
            offset_dims=(), collapsed_slice_dims=(0,), start_index_map=(0,)),
        slice_sizes=(1,),
        mode=lax.GatherScatterMode.PROMISE_IN_BOUNDS)


def _lanesum(v):
    iot = lax.iota(jnp.int32, L)
    for s in (8, 4, 2, 1):
        v = v + _take16(v, jnp.bitwise_xor(iot, s))
    return v


def _make_sc_kernel(S, V):
    RPW = S // NW
    U = 5
    NU = V // (L * U)

    mesh = plsc.VectorSubcoreMesh(core_axis_name="c", subcore_axis_name="s")

    @functools.partial(
        pl.kernel,
        mesh=mesh,
        out_type=jax.ShapeDtypeStruct((NW, L), jnp.float32),
        scratch_types=[
            pltpu.VMEM((RPW + L,), jnp.int32),
            pltpu.VMEM((RPW + L,), jnp.float32),
            pltpu.VMEM((4, V), jnp.float32),
            pltpu.VMEM((4, V), jnp.float32),
            pltpu.VMEM((L,), jnp.float32),
            pltpu.SemaphoreType.DMA,
            pltpu.SemaphoreType.DMA,
            pltpu.SemaphoreType.DMA,
            pltpu.SemaphoreType.DMA,
            pltpu.SemaphoreType.DMA,
            pltpu.SemaphoreType.DMA,
            pltpu.SemaphoreType.DMA,
            pltpu.SemaphoreType.DMA,
        ],
    )
    def sc_kernel(inp_hbm, tgt_hbm, mask_hbm, sim_hbm, out_hbm,
                  tgt_v, mask_v, sim_buf, inp_buf, res_v,
                  sem_s0, sem_s1, sem_s2, sem_s3, sem_i0, sem_i1,
                  sem_i2, sem_i3):
        wid = lax.axis_index("s") * NC + lax.axis_index("c")
        base = wid * RPW
        iotav = lax.iota(jnp.int32, L)

        pltpu.sync_copy(tgt_hbm.at[pl.ds(base, RPW)], tgt_v.at[pl.ds(0, RPW)])
        pltpu.sync_copy(mask_hbm.at[pl.ds(base, RPW)],
                        mask_v.at[pl.ds(0, RPW)])

        def scalar_at(ref, i):
            return ref[pl.ds(i, L)][0]

        sem_s = (sem_s0, sem_s1, sem_s2, sem_s3)
        sem_i = (sem_i0, sem_i1, sem_i2, sem_i3)

        def sim_copy(r, b):
            return pltpu.make_async_copy(sim_hbm.at[scalar_at(tgt_v, r)],
                                         sim_buf.at[b], sem_s[b])

        def inp_copy(r, b):
            return pltpu.make_async_copy(inp_hbm.at[base + r],
                                         inp_buf.at[b], sem_i[b])

        for b in range(4):
            sim_copy(b, b).start()
            inp_copy(b, b).start()

        def g_body(g, carry):
            acc_out, acc_ml, acc_m = carry
            for b in range(4):
                r = g * 4 + b
                sim_copy(r, b).wait()
                inp_copy(r, b).wait()

                def chunk_body(c, acc):
                    accs = list(acc)
                    off0 = c * (L * U)
                    for u in range(U):
                        off = off0 + u * L
                        s = jnp.exp(sim_buf[b, pl.ds(off, L)] * (1.0 / TAU))
                        accs[2 * u] = accs[2 * u] + s
                        accs[2 * u + 1] = (accs[2 * u + 1]
                                           + s * inp_buf[b, pl.ds(off, L)])
                    return tuple(accs)

                z = jnp.zeros((L,), jnp.float32)
                accs = lax.fori_loop(0, NU, chunk_body, (z,) * (2 * U))
                d_acc = accs[0]
                x_acc = accs[1]
                for u in range(1, U):
                    d_acc = d_acc + accs[2 * u]
                    x_acc = x_acc + accs[2 * u + 1]
                c_vec = _lanesum(x_acc) / _lanesum(d_acc)

                tgt_s = scalar_at(tgt_v, r)
                toff = (tgt_s // L) * L
                logit_vec = _lanesum(
                    jnp.where(iotav == tgt_s - toff,
                              inp_buf[b, pl.ds(toff, L)], 0.0))

                m_vec = jnp.full((L,), scalar_at(mask_v, r), jnp.float32)
                acc_out = acc_out + m_vec * c_vec
                acc_ml = acc_ml + m_vec * logit_vec
                acc_m = acc_m + m_vec

                @pl.when(r + 4 < RPW)
                def _():
                    sim_copy(r + 4, b).start()
                    inp_copy(r + 4, b).start()
            return (acc_out, acc_ml, acc_m)

        zero = jnp.zeros((L,), jnp.float32)
        acc_out, acc_ml, acc_m = lax.fori_loop(0, RPW // 4, g_body,
                                               (zero, zero, zero))

        res = jnp.where(iotav == 0, -acc_ml,
                        jnp.where(iotav == 1, -acc_out,
                                  jnp.where(iotav == 2, acc_m, 0.0)))
        res_v[...] = res
        pltpu.sync_copy(res_v, out_hbm.at[wid])

    return sc_kernel


def _make_tc_kernel(BT, V, S):
    NT = BT - S
    RB = 16
    NG = NT // RB

    NSLOT = 4

    def tc_body(tgt_ref, mask_ref, sim_hbm, inp_ref, out_ref, simscr,
                sem0, sem1, sem2, sem3):
        g = pl.program_id(0)
        sems = (sem0, sem1, sem2, sem3)

        def start_rows(gg, slot):
            for j in range(RB):
                row = tgt_ref[S + gg * RB + j]
                pltpu.make_async_copy(sim_hbm.at[row],
                                      simscr.at[slot, j],
                                      sems[slot]).start()

        def wait_rows(slot):
            pltpu.make_async_copy(sim_hbm.at[pl.ds(0, RB)],
                                  simscr.at[slot],
                                  sems[slot]).wait()

        @pl.when(g == 0)
        def _():
            out_ref[...] = jnp.zeros_like(out_ref)
            for gg in range(NSLOT - 1):
                start_rows(gg, gg)

        for p in range(NSLOT):
            @pl.when((g + NSLOT - 1 < NG) & (g % NSLOT == p))
            def _(p=p):
                start_rows(g + NSLOT - 1, (p + NSLOT - 1) % NSLOT)

        for p in range(NSLOT):
            @pl.when(g % NSLOT == p)
            def _(p=p):
                wait_rows(p)

        par = g % NSLOT
        sim_block = simscr[pl.ds(par, 1)][0]
        inp_block = inp_ref[...]
        s = jnp.exp2(sim_block * (LOG2E / TAU))
        denom = jnp.sum(s, axis=1)
        dot = jnp.sum(inp_block * s, axis=1)
        m = jnp.stack([mask_ref[S + g * RB + j] for j in range(RB)])
        tgt_col = jnp.stack([tgt_ref[S + g * RB + j]
                             for j in range(RB)])[:, None]
        lanes = lax.broadcasted_iota(jnp.int32, (RB, V), 1)
        logit = jnp.sum(jnp.where(lanes == tgt_col, inp_block, 0.0),
                        axis=1)

        ml_c = -jnp.sum(m * logit)
        out_c = -jnp.sum(m * dot / denom)
        m_c = jnp.sum(m)

        lane = lax.broadcasted_iota(jnp.int32, (8, 128), 1)
        sub = lax.broadcasted_iota(jnp.int32, (8, 128), 0)
        contrib = jnp.where(
            (sub == 0) & (lane == 0), ml_c,
            jnp.where((sub == 0) & (lane == 1), out_c,
                      jnp.where((sub == 0) & (lane == 2), m_c, 0.0)))
        out_ref[...] += contrib

    grid_spec = pltpu.PrefetchScalarGridSpec(
        num_scalar_prefetch=2,
        grid=(NG,),
        in_specs=[
            pl.BlockSpec(memory_space=pltpu.MemorySpace.HBM),
            pl.BlockSpec((RB, V), lambda g, tgt, msk: (S // RB + g, 0)),
        ],
        out_specs=pl.BlockSpec((8, 128), lambda g, tgt, msk: (0, 0)),
        scratch_shapes=[
            pltpu.VMEM((4, RB, V), jnp.float32),
            pltpu.SemaphoreType.DMA,
            pltpu.SemaphoreType.DMA,
            pltpu.SemaphoreType.DMA,
            pltpu.SemaphoreType.DMA,
        ],
    )
    return pl.pallas_call(
        tc_body,
        grid_spec=grid_spec,
        out_shape=jax.ShapeDtypeStruct((8, 128), jnp.float32),
        compiler_params=pltpu.CompilerParams(
            dimension_semantics=("arbitrary",)),
    )


@jax.jit
def kernel(input, target, mask, sim_matrix):
    B, T, V = input.shape
    BT = B * T
    inp2 = input.reshape(BT, V)
    tgt = target.reshape(BT).astype(jnp.int32)
    mask1 = mask.reshape(BT)

    sc_partials = _make_sc_kernel(S_SC, V)(inp2, tgt, mask1, sim_matrix)
    tc_partials = _make_tc_kernel(BT, V, S_SC)(tgt, mask1, sim_matrix, inp2)

    ml_sum = jnp.sum(sc_partials[:, 0]) + tc_partials[0, 0]
    out_sum = jnp.sum(sc_partials[:, 1]) + tc_partials[0, 1]
    msum = jnp.sum(sc_partials[:, 2]) + tc_partials[0, 2]
    ml_output = ml_sum / msum
    output = ALPHA * (out_sum / msum) + (1.0 - ALPHA) * ml_output
    return (ml_output, output)
